# baseline (speedup 1.0000x reference)
import functools

import jax
import jax.numpy as jnp
from jax import lax
from jax.experimental import pallas as pl
from jax.experimental.pallas import tpu as pltpu
from jax.experimental.pallas import tpu_sc as plsc

VOCAB = 100000
D = 64
B = 4096
L = 50

NC = 2
NS = 16
NW = NC * NS
B_PER_W = B // NW
ROWS_PER_CHUNK = 2
IDX_PER_CHUNK = ROWS_PER_CHUNK * L
CHUNKS = B_PER_W // ROWS_PER_CHUNK
K = 4
NG = CHUNKS // K
NLANE = 16
NCOL = D // NLANE


def _body(x_hbm, table_hbm, out_hbm, idx_v, buf_v, out_v, sem0, sem1):
    cid = lax.axis_index("c")
    sid = lax.axis_index("s")
    wid = sid * NC + cid

    pltpu.sync_copy(x_hbm.at[wid], idx_v)

    sems = (sem0, sem1)

    def issue_group(g, b):
        for kk in range(K):
            pltpu.async_copy(
                table_hbm.at[idx_v.at[g * K + kk]], buf_v.at[b, kk], sems[b]
            )

    issue_group(0, 0)
    issue_group(1, 1)

    @pl.loop(0, NG, step=2)
    def _(g0):
        for b in range(2):
            g = g0 + b
            for kk in range(K):
                pltpu.make_async_copy(
                    table_hbm.at[idx_v.at[0]], buf_v.at[b, kk], sems[b]
                ).wait()

            @pl.loop(0, K)
            def _(kk, _b=b, _g=g):
                base_row = (_g * K + kk) * ROWS_PER_CHUNK
                for r in range(ROWS_PER_CHUNK):
                    accs = [
                        buf_v[_b, kk, r * L, pl.ds(c * NLANE, NLANE)]
                        for c in range(NCOL)
                    ]
                    for j in range(1, L):
                        accs = [
                            accs[c]
                            + buf_v[_b, kk, r * L + j, pl.ds(c * NLANE, NLANE)]
                            for c in range(NCOL)
                        ]
                    for c in range(NCOL):
                        out_v[base_row + r, pl.ds(c * NLANE, NLANE)] = accs[c]

            @pl.when(g + 2 < NG)
            def _():
                issue_group(g + 2, b)

    pltpu.sync_copy(out_v, out_hbm.at[pl.ds(wid * B_PER_W, B_PER_W)])


@functools.partial(
    pl.kernel,
    out_type=jax.ShapeDtypeStruct((B, D), jnp.float32),
    mesh=plsc.VectorSubcoreMesh(core_axis_name="c", subcore_axis_name="s"),
    compiler_params=pltpu.CompilerParams(use_tc_tiling_on_sc=False),
    scratch_types=[
        pltpu.VMEM((CHUNKS, IDX_PER_CHUNK), jnp.int32),
        pltpu.VMEM((2, K, IDX_PER_CHUNK, D), jnp.float32),
        pltpu.VMEM((B_PER_W, D), jnp.float32),
        pltpu.SemaphoreType.DMA,
        pltpu.SemaphoreType.DMA,
    ],
)
def _emb_sum(x_hbm, table_hbm, out_hbm, idx_v, buf_v, out_v, sem0, sem1):
    _body(x_hbm, table_hbm, out_hbm, idx_v, buf_v, out_v, sem0, sem1)


def kernel(x, table):
    x3 = x.reshape(NW, CHUNKS, IDX_PER_CHUNK)
    return _emb_sum(x3, table)


# speedup vs baseline: 1.0010x; 1.0010x over previous
import functools

import jax
import jax.numpy as jnp
from jax import lax
from jax.experimental import pallas as pl
from jax.experimental.pallas import tpu as pltpu
from jax.experimental.pallas import tpu_sc as plsc

VOCAB = 100000
D = 64
B = 4096
L = 50

NC = 2
NS = 16
NW = NC * NS
B_PER_W = B // NW
ROWS_PER_CHUNK = 2
IDX_PER_CHUNK = ROWS_PER_CHUNK * L
CHUNKS = B_PER_W // ROWS_PER_CHUNK
K = 4
NG = CHUNKS // K
NLANE = 16
NCOL = D // NLANE


def _body(x_hbm, table_hbm, out_hbm, idx_v, buf_v, out_v, sem0, sem1):
    cid = lax.axis_index("c")
    sid = lax.axis_index("s")
    wid = sid * NC + cid

    pltpu.sync_copy(x_hbm.at[wid], idx_v)

    sems = (sem0, sem1)

    def issue_group(g, b):
        for kk in range(K):
            pltpu.async_copy(
                table_hbm.at[idx_v.at[g * K + kk]], buf_v.at[b, kk], sems[b]
            )

    issue_group(0, 0)
    issue_group(1, 1)

    @pl.loop(0, NG, step=2)
    def _(g0):
        for b in range(2):
            g = g0 + b
            for kk in range(K):
                pltpu.make_async_copy(
                    table_hbm.at[idx_v.at[0]], buf_v.at[b, kk], sems[b]
                ).wait()

            @pl.loop(0, K)
            def _(kk, _b=b, _g=g):
                base_row = (_g * K + kk) * ROWS_PER_CHUNK
                for r in range(ROWS_PER_CHUNK):
                    accs = [
                        buf_v[_b, kk, r * L, pl.ds(c * NLANE, NLANE)]
                        for c in range(NCOL)
                    ]
                    for j in range(1, L):
                        accs = [
                            accs[c]
                            + buf_v[_b, kk, r * L + j, pl.ds(c * NLANE, NLANE)]
                            for c in range(NCOL)
                        ]
                    for c in range(NCOL):
                        out_v[base_row + r, pl.ds(c * NLANE, NLANE)] = accs[c]

            @pl.when(g + 2 < NG)
            def _():
                issue_group(g + 2, b)

    pltpu.sync_copy(out_v, out_hbm.at[pl.ds(wid * B_PER_W, B_PER_W)])


@functools.partial(
    pl.kernel,
    out_type=jax.ShapeDtypeStruct((B, D), jnp.float32),
    mesh=plsc.VectorSubcoreMesh(core_axis_name="c", subcore_axis_name="s"),
    compiler_params=pltpu.CompilerParams(
        use_tc_tiling_on_sc=False,
        disable_bounds_checks=True,
        disable_semaphore_checks=True,
        skip_device_barrier=True,
    ),
    scratch_types=[
        pltpu.VMEM((CHUNKS, IDX_PER_CHUNK), jnp.int32),
        pltpu.VMEM((2, K, IDX_PER_CHUNK, D), jnp.float32),
        pltpu.VMEM((B_PER_W, D), jnp.float32),
        pltpu.SemaphoreType.DMA,
        pltpu.SemaphoreType.DMA,
    ],
)
def _emb_sum(x_hbm, table_hbm, out_hbm, idx_v, buf_v, out_v, sem0, sem1):
    _body(x_hbm, table_hbm, out_hbm, idx_v, buf_v, out_v, sem0, sem1)


def kernel(x, table):
    x3 = x.reshape(NW, CHUNKS, IDX_PER_CHUNK)
    return _emb_sum(x3, table)
